# baseline (speedup 1.0000x reference)
import functools

import jax
import jax.numpy as jnp
from jax.experimental import pallas as pl
from jax.experimental.pallas import tpu as pltpu
from jax.experimental.pallas import tpu_sc as plsc

N = 10000
E = 320000
F = 128
EDIM = 16
H = 32

NC = 2
NS = 16
NW = NC * NS
CHUNK = 125
NQT = 8

HALF = 2
EH = E // HALF
EPWH = EH // NW
E4H = EH // 4
QCHH = E4H // CHUNK
CPTH = QCHH // NQT

_SC_MESH = plsc.VectorSubcoreMesh(core_axis_name="c", subcore_axis_name="s")
_SC_PARAMS = pltpu.CompilerParams(needs_layout_passes=False,
                                  use_tc_tiling_on_sc=False)


def _node_mlp_kernel(x_ref, w1a_ref, b1_ref, u_ref):
    u_ref[...] = jnp.dot(x_ref[...], w1a_ref[...],
                         preferred_element_type=jnp.float32) + b1_ref[...]


def _edge_mlp_kernel(g4_ref, ea4_ref, w1bd_ref, w2k_ref, b2_ref,
                     m0_ref, m1_ref, m2_ref, m3_ref):
    pre = jnp.dot(ea4_ref[...], w1bd_ref[...],
                  preferred_element_type=jnp.float32)
    h4 = jax.nn.relu(g4_ref[...] + pre)
    outs = (m0_ref, m1_ref, m2_ref, m3_ref)
    for k in range(4):
        mk = jnp.dot(h4, w2k_ref[k], preferred_element_type=jnp.float32)
        outs[k][...] = jax.nn.relu(mk + b2_ref[...])


def _combine_kernel(pa_ref, pb_ref, ca_ref, cb_ref, x_ref, o_ref):
    cnt = jnp.sum(ca_ref[...], axis=0) + jnp.sum(cb_ref[...], axis=0)
    s = pa_ref[0] + pa_ref[1] + pb_ref[0] + pb_ref[1]
    inv = 1.0 / jnp.maximum(cnt, 1.0)
    o_ref[...] = s * inv.reshape(-1, 1) + x_ref[...]


def _gather_cnt_body(u_hbm, dstg_hbm, g_hbm, cnt_hbm, idx_v, cnt_v, sem):
    c = jax.lax.axis_index("c")
    s = jax.lax.axis_index("s")
    wid = s * NC + c

    def body(i_vmem, o_vmem):
        pltpu.sync_copy(u_hbm.at[i_vmem.at[0, 0]], o_vmem)

    pltpu.emit_pipeline(
        body,
        grid=(EH // CHUNK,),
        in_specs=[pl.BlockSpec((1, 1, CHUNK), lambda i: (i, 0, 0))],
        out_specs=[pl.BlockSpec((CHUNK, H), lambda i: (i, 0))],
        core_axis_name=("c", "s"),
        dimension_semantics=(pltpu.PARALLEL,),
    )(dstg_hbm, g_hbm)

    cpt = EPWH // CHUNK
    pltpu.async_copy(dstg_hbm.at[pl.ds(wid * cpt, cpt)], idx_v, sem).wait()

    @pl.loop(0, N, step=16)
    def _zero(i):
        cnt_v[pl.ds(i, 16)] = jnp.zeros((16,), jnp.float32)

    ones = jnp.ones((16,), jnp.float32)
    tail_mask = jax.lax.iota(jnp.int32, 16) >= (16 - (CHUNK % 16))

    @pl.loop(0, cpt)
    def _hist(j):
        @pl.loop(0, CHUNK - 16, step=16)
        def _full(i):
            idx = idx_v[j, 0, pl.ds(i, 16)]
            plsc.addupdate_scatter(cnt_v, [idx], ones)

        idx = idx_v[j, 0, pl.ds(CHUNK - 16, 16)]
        plsc.addupdate_scatter(cnt_v, [idx], ones, mask=tail_mask)

    pltpu.sync_copy(cnt_v, cnt_hbm.at[wid, 0])


def _scatter_body(m0_hbm, m1_hbm, m2_hbm, m3_hbm, dst4_hbm, p_hbm,
                  idx_v, buf_v, acc_sh, sem):
    c = jax.lax.axis_index("c")
    s = jax.lax.axis_index("s")
    wid = s * NC + c
    q = wid // NQT
    t = wid % NQT

    @pl.loop(0, CHUNK)
    def _zr(i):
        @pl.loop(0, F, step=16)
        def _zc(j):
            buf_v[i, pl.ds(j, 16)] = jnp.zeros((16,), jnp.float32)

    @pl.loop(0, N // NS, step=CHUNK)
    def _zacc(r):
        pltpu.sync_copy(buf_v, acc_sh.at[pl.ds(s * (N // NS) + r, CHUNK)])

    plsc.subcore_barrier()

    pltpu.async_copy(dst4_hbm.at[q, pl.ds(t * CPTH, CPTH)], idx_v, sem).wait()

    for k, mk_hbm in enumerate((m0_hbm, m1_hbm, m2_hbm, m3_hbm)):
        @pl.when(q == k)
        def _quarter(mk_hbm=mk_hbm):
            @pl.loop(0, CPTH)
            def _chunk(j):
                pltpu.sync_copy(
                    mk_hbm.at[pl.ds((t * CPTH + j) * CHUNK, CHUNK)], buf_v)
                pltpu.sync_copy(buf_v, acc_sh.at[idx_v.at[j]], add=True)

    plsc.subcore_barrier()

    pltpu.sync_copy(acc_sh.at[pl.ds(s * (N // NS), N // NS)],
                    p_hbm.at[c, pl.ds(s * (N // NS), N // NS)])


@functools.partial(
    pl.kernel,
    out_type=[jax.ShapeDtypeStruct((EH, H), jnp.float32),
              jax.ShapeDtypeStruct((NW, 1, N), jnp.float32)],
    mesh=_SC_MESH,
    compiler_params=_SC_PARAMS,
    scratch_types=[
        pltpu.VMEM((EPWH // CHUNK, 1, CHUNK), jnp.int32),
        pltpu.VMEM((N,), jnp.float32),
        pltpu.SemaphoreType.DMA,
    ],
)
def _gather_cnt(u_hbm, dstg_hbm, g_hbm, cnt_hbm, idx_v, cnt_v, sem):
    _gather_cnt_body(u_hbm, dstg_hbm, g_hbm, cnt_hbm, idx_v, cnt_v, sem)


@functools.partial(
    pl.kernel,
    out_type=jax.ShapeDtypeStruct((NC, N, F), jnp.float32),
    mesh=_SC_MESH,
    compiler_params=_SC_PARAMS,
    scratch_types=[
        pltpu.VMEM((CPTH, CHUNK), jnp.int32),
        pltpu.VMEM((CHUNK, F), jnp.float32),
        pltpu.VMEM_SHARED((N, F), jnp.float32),
        pltpu.SemaphoreType.DMA,
    ],
)
def _scatter(m0_hbm, m1_hbm, m2_hbm, m3_hbm, dst4_hbm, p_hbm,
             idx_v, buf_v, acc_sh, sem):
    _scatter_body(m0_hbm, m1_hbm, m2_hbm, m3_hbm, dst4_hbm, p_hbm,
                  idx_v, buf_v, acc_sh, sem)


def _edge_mlp(g, ea_half, w1bd, w2k, b2r):
    B4 = 1000
    g4 = g.reshape(E4H, F)
    ea4 = ea_half.reshape(E4H, 4 * EDIM)
    return pl.pallas_call(
        _edge_mlp_kernel,
        grid=(E4H // B4,),
        in_specs=[
            pl.BlockSpec((B4, F), lambda i: (i, 0)),
            pl.BlockSpec((B4, 4 * EDIM), lambda i: (i, 0)),
            pl.BlockSpec((4 * EDIM, F), lambda i: (0, 0)),
            pl.BlockSpec((4, F, F), lambda i: (0, 0, 0)),
            pl.BlockSpec((1, F), lambda i: (0, 0)),
        ],
        out_specs=[pl.BlockSpec((B4, F), lambda i: (i, 0))] * 4,
        out_shape=[jax.ShapeDtypeStruct((E4H, F), jnp.float32)] * 4,
    )(g4, ea4, w1bd, w2k, b2r)


def kernel(x, edge_index, edge_attr, W1, b1, W2, b2):
    dst = edge_index[1]
    w1a = W1[:F]
    w1b = W1[F:]
    w1bd = jax.scipy.linalg.block_diag(w1b, w1b, w1b, w1b)
    w2k = jnp.stack([jnp.pad(W2, ((k * H, F - H - k * H), (0, 0)))
                     for k in range(4)])
    b2r = b2.reshape(1, F)

    u = pl.pallas_call(
        _node_mlp_kernel,
        out_shape=jax.ShapeDtypeStruct((N, H), jnp.float32),
    )(x, w1a, b1)

    halves = []
    for h in range(HALF):
        dsth = dst[h * EH:(h + 1) * EH]
        eah = edge_attr[h * EH:(h + 1) * EH]
        g, cnt3 = _gather_cnt(u, dsth.reshape(EH // CHUNK, 1, CHUNK))
        mq = _edge_mlp(g, eah, w1bd, w2k, b2r)
        dst4 = dsth.reshape(E4H, 4).T.reshape(4, QCHH, CHUNK)
        p = _scatter(*mq, dst4)
        halves.append((p, cnt3.reshape(NW, N)))

    (pa, ca), (pb, cb) = halves
    out = pl.pallas_call(
        _combine_kernel,
        out_shape=jax.ShapeDtypeStruct((N, F), jnp.float32),
    )(pa, pb, ca, cb, x)
    return out


# speedup vs baseline: 1.2817x; 1.2817x over previous
import functools

import jax
import jax.numpy as jnp
from jax.experimental import pallas as pl
from jax.experimental.pallas import tpu as pltpu
from jax.experimental.pallas import tpu_sc as plsc

N = 10000
E = 320000
F = 128
EDIM = 16
H = 32

NC = 2
NS = 16
NW = NC * NS
CHUNK = 125
NCH = E // CHUNK
CPT = NCH // NW
NQT = 8
QCH = (E // 4) // CHUNK
QPT = QCH // NQT

_SC_MESH = plsc.VectorSubcoreMesh(core_axis_name="c", subcore_axis_name="s")
_SC_PARAMS = pltpu.CompilerParams(needs_layout_passes=False,
                                  use_tc_tiling_on_sc=False)


def _node_mlp_kernel(x_ref, w1a_ref, b1_ref, u_ref):
    u_ref[...] = jnp.dot(x_ref[...], w1a_ref[...],
                         preferred_element_type=jnp.float32) + b1_ref[...]


def _edge_mlp_kernel(g4_ref, ea4_ref, w1bd_ref, w2k_ref, b2_ref,
                     m0_ref, m1_ref, m2_ref, m3_ref):
    pre = jnp.dot(ea4_ref[...], w1bd_ref[...],
                  preferred_element_type=jnp.float32)
    h4 = jax.nn.relu(g4_ref[...] + pre)
    outs = (m0_ref, m1_ref, m2_ref, m3_ref)
    for k in range(4):
        mk = jnp.dot(h4, w2k_ref[k], preferred_element_type=jnp.float32)
        outs[k][...] = jax.nn.relu(mk + b2_ref[...])


def _combine_kernel(p_ref, cnt_ref, x_ref, o_ref):
    cnt = jnp.sum(cnt_ref[...], axis=0)
    s = p_ref[0] + p_ref[1]
    inv = 1.0 / jnp.maximum(cnt, 1.0)
    o_ref[...] = s * inv.reshape(-1, 1) + x_ref[...]


def _gather_cnt_body(u_hbm, dstg_hbm, g_hbm, cnt_hbm,
                     idx_v, gb0, gb1, cnt_v, sem0, sem1):
    c = jax.lax.axis_index("c")
    s = jax.lax.axis_index("s")
    wid = s * NC + c
    base = wid * CPT

    pltpu.sync_copy(dstg_hbm.at[pl.ds(base, CPT)], idx_v)

    def start(j, gb, sem):
        return pltpu.async_copy(u_hbm.at[idx_v.at[j, 0]], gb, sem)

    def flush(j, gb):
        pltpu.sync_copy(gb, g_hbm.at[pl.ds((base + j) * CHUNK, CHUNK)])

    start(0, gb0, sem0)

    @pl.loop(0, CPT // 2)
    def _pair(i):
        j = 2 * i
        pltpu.make_async_copy(u_hbm.at[pl.ds(0, CHUNK)], gb0, sem0).wait()
        start(j + 1, gb1, sem1)
        flush(j, gb0)
        pltpu.make_async_copy(u_hbm.at[pl.ds(0, CHUNK)], gb1, sem1).wait()

        @pl.when(j + 2 < CPT)
        def _more():
            start(j + 2, gb0, sem0)

        flush(j + 1, gb1)

    @pl.loop(0, N, step=16)
    def _zero(i):
        cnt_v[pl.ds(i, 16)] = jnp.zeros((16,), jnp.float32)

    ones = jnp.ones((16,), jnp.float32)
    tail_mask = jax.lax.iota(jnp.int32, 16) >= (16 - (CHUNK % 16))

    @pl.loop(0, CPT)
    def _hist(j):
        @pl.loop(0, CHUNK - 16, step=16)
        def _full(i):
            idx = idx_v[j, 0, pl.ds(i, 16)]
            plsc.addupdate_scatter(cnt_v, [idx], ones)

        idx = idx_v[j, 0, pl.ds(CHUNK - 16, 16)]
        plsc.addupdate_scatter(cnt_v, [idx], ones, mask=tail_mask)

    pltpu.sync_copy(cnt_v, cnt_hbm.at[wid, 0])


def _scatter_body(m0_hbm, m1_hbm, m2_hbm, m3_hbm, dst4_hbm, p_hbm,
                  idx_v, mb0, mb1, acc_sh, sem0, sem1):
    c = jax.lax.axis_index("c")
    s = jax.lax.axis_index("s")
    wid = s * NC + c
    q = wid // NQT
    t = wid % NQT

    @pl.loop(0, CHUNK)
    def _zr(i):
        @pl.loop(0, F, step=16)
        def _zc(j):
            mb0[i, pl.ds(j, 16)] = jnp.zeros((16,), jnp.float32)

    @pl.loop(0, N // NS, step=CHUNK)
    def _zacc(r):
        pltpu.sync_copy(mb0, acc_sh.at[pl.ds(s * (N // NS) + r, CHUNK)])

    plsc.subcore_barrier()

    pltpu.async_copy(dst4_hbm.at[q, pl.ds(t * QPT, QPT)], idx_v, sem0).wait()

    for k, mk_hbm in enumerate((m0_hbm, m1_hbm, m2_hbm, m3_hbm)):
        @pl.when(q == k)
        def _quarter(mk_hbm=mk_hbm):
            def start(j, mb, sem):
                pltpu.async_copy(
                    mk_hbm.at[pl.ds((t * QPT + j) * CHUNK, CHUNK)], mb, sem)

            def scat(j, mb):
                pltpu.sync_copy(mb, acc_sh.at[idx_v.at[j]], add=True)

            start(0, mb0, sem0)

            @pl.loop(0, QPT // 2)
            def _pair(i):
                j = 2 * i
                pltpu.make_async_copy(mk_hbm.at[pl.ds(0, CHUNK)], mb0,
                                      sem0).wait()
                start(j + 1, mb1, sem1)
                scat(j, mb0)
                pltpu.make_async_copy(mk_hbm.at[pl.ds(0, CHUNK)], mb1,
                                      sem1).wait()

                @pl.when(j + 2 < QPT)
                def _more():
                    start(j + 2, mb0, sem0)

                scat(j + 1, mb1)

    plsc.subcore_barrier()

    pltpu.sync_copy(acc_sh.at[pl.ds(s * (N // NS), N // NS)],
                    p_hbm.at[c, pl.ds(s * (N // NS), N // NS)])


@functools.partial(
    pl.kernel,
    out_type=[jax.ShapeDtypeStruct((E, H), jnp.float32),
              jax.ShapeDtypeStruct((NW, 1, N), jnp.float32)],
    mesh=_SC_MESH,
    compiler_params=_SC_PARAMS,
    scratch_types=[
        pltpu.VMEM((CPT, 1, CHUNK), jnp.int32),
        pltpu.VMEM((CHUNK, H), jnp.float32),
        pltpu.VMEM((CHUNK, H), jnp.float32),
        pltpu.VMEM((N,), jnp.float32),
        pltpu.SemaphoreType.DMA,
        pltpu.SemaphoreType.DMA,
    ],
)
def _gather_cnt(u_hbm, dstg_hbm, g_hbm, cnt_hbm,
                idx_v, gb0, gb1, cnt_v, sem0, sem1):
    _gather_cnt_body(u_hbm, dstg_hbm, g_hbm, cnt_hbm,
                     idx_v, gb0, gb1, cnt_v, sem0, sem1)


@functools.partial(
    pl.kernel,
    out_type=jax.ShapeDtypeStruct((NC, N, F), jnp.float32),
    mesh=_SC_MESH,
    compiler_params=_SC_PARAMS,
    scratch_types=[
        pltpu.VMEM((QPT, CHUNK), jnp.int32),
        pltpu.VMEM((CHUNK, F), jnp.float32),
        pltpu.VMEM((CHUNK, F), jnp.float32),
        pltpu.VMEM_SHARED((N, F), jnp.float32),
        pltpu.SemaphoreType.DMA,
        pltpu.SemaphoreType.DMA,
    ],
)
def _scatter(m0_hbm, m1_hbm, m2_hbm, m3_hbm, dst4_hbm, p_hbm,
             idx_v, mb0, mb1, acc_sh, sem0, sem1):
    _scatter_body(m0_hbm, m1_hbm, m2_hbm, m3_hbm, dst4_hbm, p_hbm,
                  idx_v, mb0, mb1, acc_sh, sem0, sem1)


def kernel(x, edge_index, edge_attr, W1, b1, W2, b2):
    dst = edge_index[1]
    w1a = W1[:F]
    w1b = W1[F:]
    w1bd = jax.scipy.linalg.block_diag(w1b, w1b, w1b, w1b)
    w2k = jnp.stack([jnp.pad(W2, ((k * H, F - H - k * H), (0, 0)))
                     for k in range(4)])

    u = pl.pallas_call(
        _node_mlp_kernel,
        out_shape=jax.ShapeDtypeStruct((N, H), jnp.float32),
    )(x, w1a, b1)

    g, cnt3 = _gather_cnt(u, dst.reshape(NCH, 1, CHUNK))

    E4 = E // 4
    B4 = 1000
    mq = pl.pallas_call(
        _edge_mlp_kernel,
        grid=(E4 // B4,),
        in_specs=[
            pl.BlockSpec((B4, F), lambda i: (i, 0)),
            pl.BlockSpec((B4, 4 * EDIM), lambda i: (i, 0)),
            pl.BlockSpec((4 * EDIM, F), lambda i: (0, 0)),
            pl.BlockSpec((4, F, F), lambda i: (0, 0, 0)),
            pl.BlockSpec((1, F), lambda i: (0, 0)),
        ],
        out_specs=[pl.BlockSpec((B4, F), lambda i: (i, 0))] * 4,
        out_shape=[jax.ShapeDtypeStruct((E4, F), jnp.float32)] * 4,
    )(g.reshape(E4, F), edge_attr.reshape(E4, 4 * EDIM), w1bd, w2k,
      b2.reshape(1, F))

    dst4 = dst.reshape(E4, 4).T.reshape(4, QCH, CHUNK)
    partials = _scatter(*mq, dst4)

    out = pl.pallas_call(
        _combine_kernel,
        out_shape=jax.ShapeDtypeStruct((N, F), jnp.float32),
    )(partials, cnt3.reshape(NW, N), x)
    return out


# speedup vs baseline: 1.3551x; 1.0573x over previous
import functools

import jax
import jax.numpy as jnp
from jax.experimental import pallas as pl
from jax.experimental.pallas import tpu as pltpu
from jax.experimental.pallas import tpu_sc as plsc

N = 10000
E = 320000
F = 128
EDIM = 16
H = 32

NC = 2
NS = 16
NW = NC * NS
CHUNK = 125
NCH = E // CHUNK
CPT = NCH // NW
NQT = 8
QCH = (E // 4) // CHUNK
QPT = QCH // NQT

_SC_MESH = plsc.VectorSubcoreMesh(core_axis_name="c", subcore_axis_name="s")
_SC_PARAMS = pltpu.CompilerParams(needs_layout_passes=False,
                                  use_tc_tiling_on_sc=False)


def _node_mlp_kernel(x_ref, w1a_ref, b1_ref, u_ref):
    u_ref[...] = jnp.dot(x_ref[...], w1a_ref[...],
                         preferred_element_type=jnp.float32) + b1_ref[...]


def _edge_mlp_kernel(g4_ref, ea4_ref, w1bd_ref, w2k_ref, b2_ref,
                     m0_ref, m1_ref, m2_ref, m3_ref):
    pre = jnp.dot(ea4_ref[...], w1bd_ref[...],
                  preferred_element_type=jnp.float32)
    h4 = jax.nn.relu(g4_ref[...] + pre)
    outs = (m0_ref, m1_ref, m2_ref, m3_ref)
    for k in range(4):
        mk = jnp.dot(h4, w2k_ref[k], preferred_element_type=jnp.float32)
        outs[k][...] = jax.nn.relu(mk + b2_ref[...])


def _combine_kernel(p_ref, cnt_ref, x_ref, o_ref):
    cnt = jnp.sum(cnt_ref[...], axis=0)
    s = p_ref[0] + p_ref[1]
    inv = 1.0 / jnp.maximum(cnt, 1.0)
    o_ref[...] = s * inv.reshape(-1, 1) + x_ref[...]


def _gather_cnt_body(u_hbm, dstg_hbm, g_hbm, cnt_hbm,
                     idx_v, gb0, gb1, gb2, gb3, cnt_v,
                     sem0, sem1, sem2, sem3):
    c = jax.lax.axis_index("c")
    s = jax.lax.axis_index("s")
    wid = s * NC + c
    base = wid * CPT

    pltpu.sync_copy(dstg_hbm.at[pl.ds(base, CPT)], idx_v)

    def start(j, gb, sem):
        return pltpu.async_copy(u_hbm.at[idx_v.at[j, 0]], gb, sem)

    def flush(j, gb):
        pltpu.sync_copy(gb, g_hbm.at[pl.ds((base + j) * CHUNK, CHUNK)])

    gbs = (gb0, gb1, gb2, gb3)
    sems = (sem0, sem1, sem2, sem3)
    for j in range(3):
        start(j, gbs[j], sems[j])

    @pl.loop(0, CPT // 4)
    def _quad(i):
        j4 = 4 * i
        for r in range(4):
            pltpu.make_async_copy(u_hbm.at[pl.ds(0, CHUNK)], gbs[r],
                                  sems[r]).wait()

            @pl.when(j4 + r + 3 < CPT)
            def _more(r=r):
                start(j4 + r + 3, gbs[(r + 3) % 4], sems[(r + 3) % 4])

            flush(j4 + r, gbs[r])

    @pl.loop(0, N, step=16)
    def _zero(i):
        cnt_v[pl.ds(i, 16)] = jnp.zeros((16,), jnp.float32)

    ones = jnp.ones((16,), jnp.float32)
    tail_mask = jax.lax.iota(jnp.int32, 16) >= (16 - (CHUNK % 16))

    @pl.loop(0, CPT)
    def _hist(j):
        @pl.loop(0, CHUNK - 16, step=16)
        def _full(i):
            idx = idx_v[j, 0, pl.ds(i, 16)]
            plsc.addupdate_scatter(cnt_v, [idx], ones)

        idx = idx_v[j, 0, pl.ds(CHUNK - 16, 16)]
        plsc.addupdate_scatter(cnt_v, [idx], ones, mask=tail_mask)

    pltpu.sync_copy(cnt_v, cnt_hbm.at[wid, 0])


def _scatter_body(m0_hbm, m1_hbm, m2_hbm, m3_hbm, dst4_hbm, p_hbm,
                  idx_v, mb0, mb1, acc_sh, sem0, sem1):
    c = jax.lax.axis_index("c")
    s = jax.lax.axis_index("s")
    wid = s * NC + c
    q = wid // NQT
    t = wid % NQT

    @pl.loop(0, CHUNK)
    def _zr(i):
        @pl.loop(0, F, step=16)
        def _zc(j):
            mb0[i, pl.ds(j, 16)] = jnp.zeros((16,), jnp.float32)

    @pl.loop(0, N // NS, step=CHUNK)
    def _zacc(r):
        pltpu.sync_copy(mb0, acc_sh.at[pl.ds(s * (N // NS) + r, CHUNK)])

    plsc.subcore_barrier()

    pltpu.async_copy(dst4_hbm.at[q, pl.ds(t * QPT, QPT)], idx_v, sem0).wait()

    for k, mk_hbm in enumerate((m0_hbm, m1_hbm, m2_hbm, m3_hbm)):
        @pl.when(q == k)
        def _quarter(mk_hbm=mk_hbm):
            def start(j, mb, sem):
                pltpu.async_copy(
                    mk_hbm.at[pl.ds((t * QPT + j) * CHUNK, CHUNK)], mb, sem)

            def scat(j, mb):
                pltpu.sync_copy(mb, acc_sh.at[idx_v.at[j]], add=True)

            start(0, mb0, sem0)

            @pl.loop(0, QPT // 2)
            def _pair(i):
                j = 2 * i
                pltpu.make_async_copy(mk_hbm.at[pl.ds(0, CHUNK)], mb0,
                                      sem0).wait()
                start(j + 1, mb1, sem1)
                scat(j, mb0)
                pltpu.make_async_copy(mk_hbm.at[pl.ds(0, CHUNK)], mb1,
                                      sem1).wait()

                @pl.when(j + 2 < QPT)
                def _more():
                    start(j + 2, mb0, sem0)

                scat(j + 1, mb1)

    plsc.subcore_barrier()

    pltpu.sync_copy(acc_sh.at[pl.ds(s * (N // NS), N // NS)],
                    p_hbm.at[c, pl.ds(s * (N // NS), N // NS)])


@functools.partial(
    pl.kernel,
    out_type=[jax.ShapeDtypeStruct((E, H), jnp.float32),
              jax.ShapeDtypeStruct((NW, 1, N), jnp.float32)],
    mesh=_SC_MESH,
    compiler_params=_SC_PARAMS,
    scratch_types=[
        pltpu.VMEM((CPT, 1, CHUNK), jnp.int32),
        pltpu.VMEM((CHUNK, H), jnp.float32),
        pltpu.VMEM((CHUNK, H), jnp.float32),
        pltpu.VMEM((CHUNK, H), jnp.float32),
        pltpu.VMEM((CHUNK, H), jnp.float32),
        pltpu.VMEM((N,), jnp.float32),
        pltpu.SemaphoreType.DMA,
        pltpu.SemaphoreType.DMA,
        pltpu.SemaphoreType.DMA,
        pltpu.SemaphoreType.DMA,
    ],
)
def _gather_cnt(u_hbm, dstg_hbm, g_hbm, cnt_hbm,
                idx_v, gb0, gb1, gb2, gb3, cnt_v, sem0, sem1, sem2, sem3):
    _gather_cnt_body(u_hbm, dstg_hbm, g_hbm, cnt_hbm,
                     idx_v, gb0, gb1, gb2, gb3, cnt_v,
                     sem0, sem1, sem2, sem3)


@functools.partial(
    pl.kernel,
    out_type=jax.ShapeDtypeStruct((NC, N, F), jnp.float32),
    mesh=_SC_MESH,
    compiler_params=_SC_PARAMS,
    scratch_types=[
        pltpu.VMEM((QPT, CHUNK), jnp.int32),
        pltpu.VMEM((CHUNK, F), jnp.float32),
        pltpu.VMEM((CHUNK, F), jnp.float32),
        pltpu.VMEM_SHARED((N, F), jnp.float32),
        pltpu.SemaphoreType.DMA,
        pltpu.SemaphoreType.DMA,
    ],
)
def _scatter(m0_hbm, m1_hbm, m2_hbm, m3_hbm, dst4_hbm, p_hbm,
             idx_v, mb0, mb1, acc_sh, sem0, sem1):
    _scatter_body(m0_hbm, m1_hbm, m2_hbm, m3_hbm, dst4_hbm, p_hbm,
                  idx_v, mb0, mb1, acc_sh, sem0, sem1)


def kernel(x, edge_index, edge_attr, W1, b1, W2, b2):
    dst = edge_index[1]
    w1a = W1[:F]
    w1b = W1[F:]
    w1bd = jax.scipy.linalg.block_diag(w1b, w1b, w1b, w1b)
    w2k = jnp.stack([jnp.pad(W2, ((k * H, F - H - k * H), (0, 0)))
                     for k in range(4)])

    u = pl.pallas_call(
        _node_mlp_kernel,
        out_shape=jax.ShapeDtypeStruct((N, H), jnp.float32),
    )(x, w1a, b1)

    g, cnt3 = _gather_cnt(u, dst.reshape(NCH, 1, CHUNK))

    E4 = E // 4
    B4 = 1000
    mq = pl.pallas_call(
        _edge_mlp_kernel,
        grid=(E4 // B4,),
        in_specs=[
            pl.BlockSpec((B4, F), lambda i: (i, 0)),
            pl.BlockSpec((B4, 4 * EDIM), lambda i: (i, 0)),
            pl.BlockSpec((4 * EDIM, F), lambda i: (0, 0)),
            pl.BlockSpec((4, F, F), lambda i: (0, 0, 0)),
            pl.BlockSpec((1, F), lambda i: (0, 0)),
        ],
        out_specs=[pl.BlockSpec((B4, F), lambda i: (i, 0))] * 4,
        out_shape=[jax.ShapeDtypeStruct((E4, F), jnp.float32)] * 4,
    )(g.reshape(E4, F), edge_attr.reshape(E4, 4 * EDIM), w1bd, w2k,
      b2.reshape(1, F))

    dst4 = jnp.stack([dst[k::4] for k in range(4)]).reshape(4, QCH, CHUNK)
    partials = _scatter(*mq, dst4)

    out = pl.pallas_call(
        _combine_kernel,
        out_shape=jax.ShapeDtypeStruct((N, F), jnp.float32),
    )(partials, cnt3.reshape(NW, N), x)
    return out
